# baseline (speedup 1.0000x reference)
import jax
import jax.numpy as jnp
from jax import lax
from jax.experimental import pallas as pl
from jax.experimental.pallas import tpu as pltpu
from jax.experimental.pallas import tpu_sc as plsc

N = 100000
D = 128
G = 512
L = 16
NC = 2
NS = 16
NW = NC * NS
BASE_ROWS = 3120
EXTRA_W = 10
CHUNK = 80
NCHUNK = BASE_ROWS // CHUNK
GPC = CHUNK // L
CLAMP = 60.0

_mesh = plsc.VectorSubcoreMesh(core_axis_name="c", subcore_axis_name="s")


def _wid_base():
    c = lax.axis_index("c")
    s = lax.axis_index("s")
    w = s * NC + c
    base = w * BASE_ROWS + L * jnp.minimum(w, EXTRA_W)
    nex = jnp.where(w < EXTRA_W, 1, 0)
    return c, s, w, base, nex


def _lane_iota():
    return lax.broadcasted_iota(jnp.int32, (L,), 0)


def _take(v, idx):
    return v.at[idx].get(mode="promise_in_bounds", unique_indices=False)


def _bsum(v):
    lane = _lane_iota()
    for k in (8, 4, 2, 1):
        v = v + _take(v, lane ^ k)
    return v


def _ka_body(x_hbm, batch_hbm, w_hbm,
             e_hbm, wsum_hbm, parts_hbm,
             xbuf, xbuf2, sem0, sem1, wbuf, lbuf, bbuf, acc, stage, idxbuf,
             shared):
    c, s, w, base, nex = _wid_base()
    lane = _lane_iota()
    cols = [lane + L * j for j in range(8)]

    pltpu.sync_copy(w_hbm, wbuf)
    wregs = [wbuf[pl.ds(L * j, L)] for j in range(8)]

    pltpu.sync_copy(batch_hbm.at[pl.ds(base, BASE_ROWS)],
                    bbuf.at[pl.ds(0, BASE_ROWS)])

    def exin(_, z):
        pltpu.sync_copy(batch_hbm.at[pl.ds(base + BASE_ROWS, L)],
                        bbuf.at[pl.ds(BASE_ROWS, L)])
        return z

    lax.fori_loop(0, nex, exin, 0)

    zv = jnp.zeros((L,), jnp.float32)

    def zbody(i, z):
        for j in range(8):
            acc[i, pl.ds(L * j, L)] = zv
        return z

    lax.fori_loop(0, G, zbody, 0)

    @pl.when(s == 0)
    def _():
        pltpu.sync_copy(acc, shared)

    def group(buf, row0_buf, goff, sv):
        segv = bbuf[pl.ds(goff, L)]
        seg0 = segv[0]
        seg15 = segv[15]
        evec = jnp.zeros((L,), jnp.float32)
        for r in range(16):
            row = row0_buf + r
            p = buf[row, pl.ds(0, L)] * wregs[0]
            for j in range(1, 8):
                p = p + buf[row, pl.ds(L * j, L)] * wregs[j]
            p = _bsum(p)
            er = jnp.exp(jnp.minimum(jnp.maximum(p, -CLAMP), CLAMP))
            evec = jnp.where(lane == r, er, evec)
        lbuf[pl.ds(goff, L)] = evec

        def uniform():
            tot = [None] * 8
            for r in range(16):
                es = _take(evec, jnp.full((L,), r, jnp.int32))
                row = row0_buf + r
                for j in range(8):
                    prod = buf[row, pl.ds(L * j, L)] * es
                    tot[j] = prod if r == 0 else tot[j] + prod
            for j in range(8):
                sl = pl.ds(L * j, L)
                acc[seg0, sl] = acc[seg0, sl] + tot[j]

        def mixed():
            for r in range(16):
                es = _take(evec, jnp.full((L,), r, jnp.int32))
                sr = segv[r]
                row = row0_buf + r
                for j in range(8):
                    sl = pl.ds(L * j, L)
                    acc[sr, sl] = acc[sr, sl] + buf[row, sl] * es

        lax.cond(seg0 == seg15, uniform, mixed)
        return sv + evec

    def start(ci, buf, sem):
        pltpu.async_copy(x_hbm.at[pl.ds(base + ci * CHUNK, CHUNK), :],
                         buf, sem)

    def wait(buf, sem):
        pltpu.make_async_copy(x_hbm.at[pl.ds(base, CHUNK), :],
                              buf, sem).wait()

    def process(buf, ci, sv):
        def gbody(g, sv2):
            return group(buf, g * L, ci * CHUNK + g * L, sv2)

        return lax.fori_loop(0, GPC, gbody, sv)

    start(0, xbuf, sem0)

    def pair_body(i, sv):
        start(2 * i + 1, xbuf2, sem1)
        wait(xbuf, sem0)
        sv = process(xbuf, 2 * i, sv)
        start(2 * i + 2, xbuf, sem0)
        wait(xbuf2, sem1)
        return process(xbuf2, 2 * i + 1, sv)

    svec = lax.fori_loop(0, (NCHUNK - 1) // 2, pair_body,
                         jnp.zeros((L,), jnp.float32))
    wait(xbuf, sem0)
    svec = process(xbuf, NCHUNK - 1, svec)

    def exbody(_, sv):
        pltpu.sync_copy(x_hbm.at[pl.ds(base + BASE_ROWS, L), :],
                        xbuf.at[pl.ds(0, L), :])
        return group(xbuf, 0, BASE_ROWS, sv)

    svec = lax.fori_loop(0, nex, exbody, svec)

    stage[...] = _bsum(svec)
    pltpu.sync_copy(stage, wsum_hbm.at[w])

    pltpu.sync_copy(lbuf.at[pl.ds(0, BASE_ROWS)],
                    e_hbm.at[pl.ds(base, BASE_ROWS)])

    def exw(_, z):
        pltpu.sync_copy(lbuf.at[pl.ds(BASE_ROWS, L)],
                        e_hbm.at[pl.ds(base + BASE_ROWS, L)])
        return z

    lax.fori_loop(0, nex, exw, 0)

    seg_lo = bbuf[pl.ds(0, L)][0]
    last_off = BASE_ROWS - L + L * nex
    seg_hi = bbuf[pl.ds(last_off, L)][15]
    plsc.subcore_barrier()
    nwin = (seg_hi - seg_lo + 128) // 128

    def wbody(k, z):
        start = jnp.minimum(seg_lo + 128 * k, G - 128)
        sb = jnp.broadcast_to(start, (L,))
        for m in range(8):
            idxbuf[pl.ds(L * m, L)] = sb + cols[m]
        pltpu.sync_copy(acc.at[pl.ds(start, 128), :],
                        shared.at[idxbuf], add=True)
        return z

    lax.fori_loop(0, nwin, wbody, 0)
    plsc.subcore_barrier()

    @pl.when(s == 0)
    def _():
        pltpu.sync_copy(shared, parts_hbm.at[c])


_ka = pl.kernel(
    _ka_body,
    out_type=(
        jax.ShapeDtypeStruct((N,), jnp.float32),
        jax.ShapeDtypeStruct((NW, L), jnp.float32),
        jax.ShapeDtypeStruct((NC, G, D), jnp.float32),
    ),
    mesh=_mesh,
    scratch_types=[
        pltpu.VMEM((CHUNK, D), jnp.float32),
        pltpu.VMEM((CHUNK, D), jnp.float32),
        pltpu.SemaphoreType.DMA,
        pltpu.SemaphoreType.DMA,
        pltpu.VMEM((D,), jnp.float32),
        pltpu.VMEM((BASE_ROWS + L,), jnp.float32),
        pltpu.VMEM((BASE_ROWS + L,), jnp.int32),
        pltpu.VMEM((G, D), jnp.float32),
        pltpu.VMEM((L,), jnp.float32),
        pltpu.VMEM((128,), jnp.int32),
        pltpu.VMEM_SHARED((G, D), jnp.float32),
    ],
)


def _kb_body(e_hbm, wsum_hbm, parts_hbm, out_hbm, alpha_hbm,
             p0, p1, obuf, ebuf, sbuf):
    c, s, w, base, nex = _wid_base()
    pltpu.sync_copy(wsum_hbm, sbuf)
    gsum = sbuf[0, :]
    for j in range(1, NW):
        gsum = gsum + sbuf[j, :]
    inv = 1.0 / gsum

    pltpu.sync_copy(parts_hbm.at[0, pl.ds(L * w, L), :], p0)
    pltpu.sync_copy(parts_hbm.at[1, pl.ds(L * w, L), :], p1)
    for r in range(16):
        for j in range(8):
            sl = pl.ds(L * j, L)
            obuf[r, sl] = (p0[r, sl] + p1[r, sl]) * inv
    pltpu.sync_copy(obuf, out_hbm.at[pl.ds(L * w, L), :])

    pltpu.sync_copy(e_hbm.at[pl.ds(base, BASE_ROWS)],
                    ebuf.at[pl.ds(0, BASE_ROWS)])

    def exin(_, z):
        pltpu.sync_copy(e_hbm.at[pl.ds(base + BASE_ROWS, L)],
                        ebuf.at[pl.ds(BASE_ROWS, L)])
        return z

    lax.fori_loop(0, nex, exin, 0)

    def gb(g, z):
        sl = pl.ds(L * g, L)
        ebuf[sl] = ebuf[sl] * inv
        return z

    lax.fori_loop(0, BASE_ROWS // L + nex, gb, 0)

    pltpu.sync_copy(ebuf.at[pl.ds(0, BASE_ROWS)],
                    alpha_hbm.at[pl.ds(base, BASE_ROWS)])

    def exw(_, z):
        pltpu.sync_copy(ebuf.at[pl.ds(BASE_ROWS, L)],
                        alpha_hbm.at[pl.ds(base + BASE_ROWS, L)])
        return z

    lax.fori_loop(0, nex, exw, 0)


_kb = pl.kernel(
    _kb_body,
    out_type=(
        jax.ShapeDtypeStruct((G, D), jnp.float32),
        jax.ShapeDtypeStruct((N,), jnp.float32),
    ),
    mesh=_mesh,
    scratch_types=[
        pltpu.VMEM((L, D), jnp.float32),
        pltpu.VMEM((L, D), jnp.float32),
        pltpu.VMEM((L, D), jnp.float32),
        pltpu.VMEM((BASE_ROWS + L,), jnp.float32),
        pltpu.VMEM((NW, L), jnp.float32),
    ],
)


@jax.jit
def kernel(x, batch, W, b):
    del b
    wvec = W.reshape(D).astype(jnp.float32)
    batch32 = batch.astype(jnp.int32)
    e, wsum, parts = _ka(x, batch32, wvec)
    out, alpha = _kb(e, wsum, parts)
    return out, alpha.reshape(N, 1)


# speedup vs baseline: 1.0411x; 1.0411x over previous
import jax
import jax.numpy as jnp
from jax import lax
from jax.experimental import pallas as pl
from jax.experimental.pallas import tpu as pltpu
from jax.experimental.pallas import tpu_sc as plsc

N = 100000
D = 128
G = 512
L = 16
NC = 2
NS = 16
NW = NC * NS
BASE_ROWS = 3120
EXTRA_W = 10
CHUNK = 80
NCHUNK = BASE_ROWS // CHUNK
GPC = CHUNK // L
CLAMP = 60.0

_mesh = plsc.VectorSubcoreMesh(core_axis_name="c", subcore_axis_name="s")


def _wid_base():
    c = lax.axis_index("c")
    s = lax.axis_index("s")
    w = s * NC + c
    base = w * BASE_ROWS + L * jnp.minimum(w, EXTRA_W)
    nex = jnp.where(w < EXTRA_W, 1, 0)
    return c, s, w, base, nex


def _lane_iota():
    return lax.broadcasted_iota(jnp.int32, (L,), 0)


def _take(v, idx):
    return v.at[idx].get(mode="promise_in_bounds", unique_indices=False)


def _bsum(v):
    lane = _lane_iota()
    for k in (8, 4, 2, 1):
        v = v + _take(v, lane ^ k)
    return v


def _ka_body(x_hbm, batch_hbm, w_hbm,
             e_hbm, wsum_hbm, parts_hbm,
             xbuf, xbuf2, sem0, sem1, wbuf, lbuf, bbuf, acc, stage, idxbuf,
             shared):
    c, s, w, base, nex = _wid_base()
    lane = _lane_iota()
    cols = [lane + L * j for j in range(8)]

    pltpu.sync_copy(w_hbm, wbuf)
    wregs = [wbuf[pl.ds(L * j, L)] for j in range(8)]

    pltpu.sync_copy(batch_hbm.at[pl.ds(base, BASE_ROWS)],
                    bbuf.at[pl.ds(0, BASE_ROWS)])

    def exin(_, z):
        pltpu.sync_copy(batch_hbm.at[pl.ds(base + BASE_ROWS, L)],
                        bbuf.at[pl.ds(BASE_ROWS, L)])
        return z

    lax.fori_loop(0, nex, exin, 0)

    zv = jnp.zeros((L,), jnp.float32)

    def zbody(i, z):
        for j in range(8):
            acc[i, pl.ds(L * j, L)] = zv
        return z

    lax.fori_loop(0, G, zbody, 0)

    @pl.when(s == 0)
    def _():
        pltpu.sync_copy(acc, shared)

    def group(buf, row0_buf, goff, sv):
        segv = bbuf[pl.ds(goff, L)]
        seg0 = segv[0]
        seg15 = segv[15]
        evec = jnp.zeros((L,), jnp.float32)
        tot = [None] * 8
        for r in range(16):
            row = row0_buf + r
            xv = [buf[row, pl.ds(L * j, L)] for j in range(8)]
            p = xv[0] * wregs[0]
            for j in range(1, 8):
                p = p + xv[j] * wregs[j]
            p = _bsum(p)
            er = jnp.exp(jnp.minimum(jnp.maximum(p, -CLAMP), CLAMP))
            evec = jnp.where(lane == r, er, evec)
            for j in range(8):
                prod = xv[j] * er
                tot[j] = prod if r == 0 else tot[j] + prod
        lbuf[pl.ds(goff, L)] = evec

        def uniform():
            for j in range(8):
                sl = pl.ds(L * j, L)
                acc[seg0, sl] = acc[seg0, sl] + tot[j]

        def mixed():
            for r in range(16):
                es = _take(evec, jnp.full((L,), r, jnp.int32))
                sr = segv[r]
                row = row0_buf + r
                for j in range(8):
                    sl = pl.ds(L * j, L)
                    acc[sr, sl] = acc[sr, sl] + buf[row, sl] * es

        lax.cond(seg0 == seg15, uniform, mixed)
        return sv + evec

    def start(ci, buf, sem):
        pltpu.async_copy(x_hbm.at[pl.ds(base + ci * CHUNK, CHUNK), :],
                         buf, sem)

    def wait(buf, sem):
        pltpu.make_async_copy(x_hbm.at[pl.ds(base, CHUNK), :],
                              buf, sem).wait()

    def process(buf, ci, sv):
        def gbody(g, sv2):
            return group(buf, g * L, ci * CHUNK + g * L, sv2)

        return lax.fori_loop(0, GPC, gbody, sv)

    start(0, xbuf, sem0)

    def pair_body(i, sv):
        start(2 * i + 1, xbuf2, sem1)
        wait(xbuf, sem0)
        sv = process(xbuf, 2 * i, sv)
        start(2 * i + 2, xbuf, sem0)
        wait(xbuf2, sem1)
        return process(xbuf2, 2 * i + 1, sv)

    svec = lax.fori_loop(0, (NCHUNK - 1) // 2, pair_body,
                         jnp.zeros((L,), jnp.float32))
    wait(xbuf, sem0)
    svec = process(xbuf, NCHUNK - 1, svec)

    def exbody(_, sv):
        pltpu.sync_copy(x_hbm.at[pl.ds(base + BASE_ROWS, L), :],
                        xbuf.at[pl.ds(0, L), :])
        return group(xbuf, 0, BASE_ROWS, sv)

    svec = lax.fori_loop(0, nex, exbody, svec)

    stage[...] = _bsum(svec)
    pltpu.sync_copy(stage, wsum_hbm.at[w])

    pltpu.sync_copy(lbuf.at[pl.ds(0, BASE_ROWS)],
                    e_hbm.at[pl.ds(base, BASE_ROWS)])

    def exw(_, z):
        pltpu.sync_copy(lbuf.at[pl.ds(BASE_ROWS, L)],
                        e_hbm.at[pl.ds(base + BASE_ROWS, L)])
        return z

    lax.fori_loop(0, nex, exw, 0)

    seg_lo = bbuf[pl.ds(0, L)][0]
    last_off = BASE_ROWS - L + L * nex
    seg_hi = bbuf[pl.ds(last_off, L)][15]
    plsc.subcore_barrier()
    nwin = (seg_hi - seg_lo + 128) // 128

    def wbody(k, z):
        start = jnp.minimum(seg_lo + 128 * k, G - 128)
        sb = jnp.broadcast_to(start, (L,))
        for m in range(8):
            idxbuf[pl.ds(L * m, L)] = sb + cols[m]
        pltpu.sync_copy(acc.at[pl.ds(start, 128), :],
                        shared.at[idxbuf], add=True)
        return z

    lax.fori_loop(0, nwin, wbody, 0)
    plsc.subcore_barrier()

    @pl.when(s == 0)
    def _():
        pltpu.sync_copy(shared, parts_hbm.at[c])


_ka = pl.kernel(
    _ka_body,
    out_type=(
        jax.ShapeDtypeStruct((N,), jnp.float32),
        jax.ShapeDtypeStruct((NW, L), jnp.float32),
        jax.ShapeDtypeStruct((NC, G, D), jnp.float32),
    ),
    mesh=_mesh,
    scratch_types=[
        pltpu.VMEM((CHUNK, D), jnp.float32),
        pltpu.VMEM((CHUNK, D), jnp.float32),
        pltpu.SemaphoreType.DMA,
        pltpu.SemaphoreType.DMA,
        pltpu.VMEM((D,), jnp.float32),
        pltpu.VMEM((BASE_ROWS + L,), jnp.float32),
        pltpu.VMEM((BASE_ROWS + L,), jnp.int32),
        pltpu.VMEM((G, D), jnp.float32),
        pltpu.VMEM((L,), jnp.float32),
        pltpu.VMEM((128,), jnp.int32),
        pltpu.VMEM_SHARED((G, D), jnp.float32),
    ],
)


ABLK = 8192


def _kb_body(e_ref, wsum_ref, parts_ref, out_ref, alpha_ref):
    inv = 1.0 / jnp.sum(wsum_ref[:, 0])
    alpha_ref[...] = e_ref[...] * inv
    out_ref[...] = (parts_ref[0] + parts_ref[1]) * inv


_kb = pl.pallas_call(
    _kb_body,
    grid=(pl.cdiv(N, ABLK),),
    in_specs=[
        pl.BlockSpec((ABLK,), lambda i: (i,)),
        pl.BlockSpec((NW, L), lambda i: (0, 0)),
        pl.BlockSpec((NC, G, D), lambda i: (0, 0, 0)),
    ],
    out_specs=(
        pl.BlockSpec((G, D), lambda i: (0, 0)),
        pl.BlockSpec((ABLK,), lambda i: (i,)),
    ),
    out_shape=(
        jax.ShapeDtypeStruct((G, D), jnp.float32),
        jax.ShapeDtypeStruct((N,), jnp.float32),
    ),
)


@jax.jit
def kernel(x, batch, W, b):
    del b
    wvec = W.reshape(D).astype(jnp.float32)
    batch32 = batch.astype(jnp.int32)
    e, wsum, parts = _ka(x, batch32, wvec)
    out, alpha = _kb(e, wsum, parts)
    return out, alpha.reshape(N, 1)
